# baseline (speedup 1.0000x reference)
import functools

import jax
import jax.numpy as jnp
from jax import lax
from jax.experimental import pallas as pl
from jax.experimental.pallas import tpu as pltpu
from jax.experimental.pallas import tpu_sc as plsc

B = 16384
D = 64
L = 16
NC, NS = 2, 16
NW = NC * NS
BPW = B // NW
CHUNK = 128
NCH = BPW // CHUNK


def _body(users_hbm, items_hbm, uf_hbm, if_hbm, ub_hbm, ib_hbm, out_hbm,
          uidx_v, iidx_v, urows_v, irows_v, bu_v, bi_v, t_v, out_v, sem):
    wid = lax.axis_index("s") * NC + lax.axis_index("c")

    pltpu.sync_copy(users_hbm.at[pl.ds(wid * NCH, NCH)], uidx_v)
    pltpu.sync_copy(items_hbm.at[pl.ds(wid * NCH, NCH)], iidx_v)

    copies = []
    for j in range(NCH):
        copies.append(pltpu.async_copy(uf_hbm.at[uidx_v.at[j]], urows_v.at[j], sem))
        copies.append(pltpu.async_copy(if_hbm.at[iidx_v.at[j]], irows_v.at[j], sem))
        copies.append(pltpu.async_copy(ub_hbm.at[uidx_v.at[j]], bu_v.at[j], sem))
        copies.append(pltpu.async_copy(ib_hbm.at[iidx_v.at[j]], bi_v.at[j], sem))
    for c in copies:
        c.wait()

    lanes = lax.iota(jnp.int32, L)

    def pass1(b, carry):
        j = b // CHUNK
        r = b % CHUNK
        acc = (urows_v[j, r, pl.ds(0, L)] * irows_v[j, r, pl.ds(0, L)]
               + urows_v[j, r, pl.ds(L, L)] * irows_v[j, r, pl.ds(L, L)])
        acc = acc + (urows_v[j, r, pl.ds(2 * L, L)] * irows_v[j, r, pl.ds(2 * L, L)]
                     + urows_v[j, r, pl.ds(3 * L, L)] * irows_v[j, r, pl.ds(3 * L, L)])
        plsc.store_scatter(t_v, [lanes * BPW + b], acc)
        return carry

    lax.fori_loop(0, BPW, pass1, 0, unroll=4)

    def pass2(g, carry):
        base = g * L
        acc = t_v[0, pl.ds(base, L)]
        for lane in range(1, L):
            acc = acc + t_v[lane, pl.ds(base, L)]
        j = g // (CHUNK // L)
        r = (g % (CHUNK // L)) * L
        acc = acc + bu_v[j, pl.ds(r, L)] + bi_v[j, pl.ds(r, L)]
        out_v[pl.ds(base, L)] = acc
        return carry

    lax.fori_loop(0, BPW // L, pass2, 0, unroll=2)

    pltpu.sync_copy(out_v, out_hbm.at[pl.ds(wid * BPW, BPW)])


@functools.partial(jax.jit, static_argnums=())
def _run(users2, items2, user_factors, item_factors, ub_flat, ib_flat):
    mesh = plsc.VectorSubcoreMesh(core_axis_name="c", subcore_axis_name="s")
    fn = pl.kernel(
        _body,
        out_type=jax.ShapeDtypeStruct((B,), jnp.float32),
        mesh=mesh,
        compiler_params=pltpu.CompilerParams(
            needs_layout_passes=False, use_tc_tiling_on_sc=False),
        scratch_types=[
            pltpu.VMEM((NCH, CHUNK), jnp.int32),
            pltpu.VMEM((NCH, CHUNK), jnp.int32),
            pltpu.VMEM((NCH, CHUNK, D), jnp.float32),
            pltpu.VMEM((NCH, CHUNK, D), jnp.float32),
            pltpu.VMEM((NCH, CHUNK), jnp.float32),
            pltpu.VMEM((NCH, CHUNK), jnp.float32),
            pltpu.VMEM((L * BPW,), jnp.float32),
            pltpu.VMEM((BPW,), jnp.float32),
            pltpu.SemaphoreType.DMA,
        ],
    )
    return fn(users2, items2, user_factors, item_factors, ub_flat, ib_flat)


def kernel(users, items, user_factors, item_factors, user_biases, item_biases):
    users2 = users.astype(jnp.int32).reshape(B // CHUNK, CHUNK)
    items2 = items.astype(jnp.int32).reshape(B // CHUNK, CHUNK)
    ub_flat = user_biases.reshape(-1)
    ib_flat = item_biases.reshape(-1)
    return _run(users2, items2, user_factors, item_factors, ub_flat, ib_flat)


# speedup vs baseline: 1.3977x; 1.3977x over previous
import jax
import jax.numpy as jnp
from jax import lax
from jax.experimental import pallas as pl
from jax.experimental.pallas import tpu as pltpu
from jax.experimental.pallas import tpu_sc as plsc

B = 16384
D = 64
L = 16
NC, NS = 2, 16
NW = NC * NS
BPW = B // NW
CHUNK = 128
NCH = BPW // CHUNK


def _body(users_hbm, items_hbm, uf_hbm, if_hbm, ub_hbm, ib_hbm, out_hbm,
          uidx_v, iidx_v, urows_v, irows_v, bu_v, bi_v, t_v,
          out_v, sem):
    wid = lax.axis_index("s") * NC + lax.axis_index("c")

    pltpu.sync_copy(users_hbm.at[pl.ds(wid * NCH, NCH)], uidx_v)
    pltpu.sync_copy(items_hbm.at[pl.ds(wid * NCH, NCH)], iidx_v)

    bias_copies = []
    for j in range(NCH):
        bias_copies.append(pltpu.async_copy(ub_hbm.at[uidx_v.at[j]], bu_v.at[j], sem))
        bias_copies.append(pltpu.async_copy(ib_hbm.at[iidx_v.at[j]], bi_v.at[j], sem))

    lanes = lax.iota(jnp.int32, L)

    for h in range(2):
        for jj in range(NCH // 2):
            j = h * (NCH // 2) + jj
            def fire(g, carry, j=j, jj=jj):
                rbase = g * L
                uv = uidx_v[j, pl.ds(rbase, L)]
                iv = iidx_v[j, pl.ds(rbase, L)]
                for k in range(L):
                    ui = jnp.sum(jnp.where(lanes == k, uv, 0))
                    ii = jnp.sum(jnp.where(lanes == k, iv, 0))
                    row = jj * CHUNK + rbase + k
                    pltpu.async_copy(uf_hbm.at[ui], urows_v.at[row], sem)
                    pltpu.async_copy(if_hbm.at[ii], irows_v.at[row], sem)
                return carry
            lax.fori_loop(0, CHUNK // L, fire, 0)

        pltpu.make_async_copy(uf_hbm.at[pl.ds(0, BPW // 2)], urows_v, sem).wait()
        pltpu.make_async_copy(if_hbm.at[pl.ds(0, BPW // 2)], irows_v, sem).wait()

        def pass1(i, carry, h=h):
            acc = (urows_v[i, pl.ds(0, L)] * irows_v[i, pl.ds(0, L)]
                   + urows_v[i, pl.ds(L, L)] * irows_v[i, pl.ds(L, L)])
            acc = acc + (urows_v[i, pl.ds(2 * L, L)] * irows_v[i, pl.ds(2 * L, L)]
                         + urows_v[i, pl.ds(3 * L, L)] * irows_v[i, pl.ds(3 * L, L)])
            plsc.store_scatter(t_v, [lanes * BPW + h * (BPW // 2) + i], acc)
            return carry

        lax.fori_loop(0, BPW // 2, pass1, 0, unroll=4)

    for c in bias_copies:
        c.wait()

    def pass2(g, carry):
        gbase = g * L
        acc = t_v[pl.ds(gbase, L)]
        for lane in range(1, L):
            acc = acc + t_v[pl.ds(lane * BPW + gbase, L)]
        j = g // (CHUNK // L)
        r = (g % (CHUNK // L)) * L
        acc = acc + bu_v[j, pl.ds(r, L)] + bi_v[j, pl.ds(r, L)]
        out_v[pl.ds(gbase, L)] = acc
        return carry

    lax.fori_loop(0, BPW // L, pass2, 0, unroll=2)

    pltpu.sync_copy(out_v, out_hbm.at[pl.ds(wid * BPW, BPW)])


@jax.jit
def _run(users2, items2, user_factors, item_factors, ub_flat, ib_flat):
    mesh = plsc.VectorSubcoreMesh(core_axis_name="c", subcore_axis_name="s")
    fn = pl.kernel(
        _body,
        out_type=jax.ShapeDtypeStruct((B,), jnp.float32),
        mesh=mesh,
        compiler_params=pltpu.CompilerParams(needs_layout_passes=False),
        scratch_types=[
            pltpu.VMEM((NCH, CHUNK), jnp.int32),
            pltpu.VMEM((NCH, CHUNK), jnp.int32),
            pltpu.VMEM((BPW // 2, D), jnp.float32),
            pltpu.VMEM((BPW // 2, D), jnp.float32),
            pltpu.VMEM((NCH, CHUNK), jnp.float32),
            pltpu.VMEM((NCH, CHUNK), jnp.float32),
            pltpu.VMEM((L * BPW,), jnp.float32),
            pltpu.VMEM((BPW,), jnp.float32),
            pltpu.SemaphoreType.DMA,
        ],
    )
    return fn(users2, items2, user_factors, item_factors, ub_flat, ib_flat)


def kernel(users, items, user_factors, item_factors, user_biases, item_biases):
    users2 = users.astype(jnp.int32).reshape(B // CHUNK, CHUNK)
    items2 = items.astype(jnp.int32).reshape(B // CHUNK, CHUNK)
    return _run(users2, items2, user_factors, item_factors,
                user_biases.reshape(-1), item_biases.reshape(-1))
